# baseline (speedup 1.0000x reference)
import functools

import jax
import jax.numpy as jnp
from jax import lax
from jax.experimental import pallas as pl
from jax.experimental.pallas import tpu as pltpu
from jax.experimental.pallas import tpu_sc as plsc

def _lane_allsum(v):
    idx = lax.iota(jnp.int32, 16)
    for sh in (8, 4, 2, 1):
        v = v + v.at[idx ^ sh].get(mode="promise_in_bounds")
    return v


L = 16
NG = 5
ROWS = 77
RPAD = 80
NO = 90


def _body(x_hbm, w_hbm, a_hbm, out_hbm,
          attn_v, w_v, xslab_v, x10_v, vv_v, st_v, est_v, out77_v,
          vv_sh, e_sh):
    c = lax.axis_index("c")
    tid = lax.axis_index("s")

    @pl.when(c == 0)
    def _core0():
        @pl.when(tid < NG)
        def _phase_a():
            g = tid
            pltpu.sync_copy(a_hbm.at[0], attn_v.at[pl.ds(0, 2 * NO)])
            pltpu.sync_copy(w_hbm, w_v)

            acc1 = jnp.zeros((L,), jnp.float32)
            acc2 = jnp.zeros((L,), jnp.float32)
            for o in range(NO):
                wrow = w_v[o, pl.ds(g * L, L)]
                c1 = attn_v[pl.ds(o, L)][0]
                c2 = attn_v[pl.ds(NO + o, L)][0]
                acc1 = acc1 + c1 * wrow
                acc2 = acc2 + c2 * wrow
            st_v[...] = acc1
            pltpu.sync_copy(st_v, vv_sh.at[pl.ds(tid * L, L)])
            st_v[...] = acc2
            pltpu.sync_copy(st_v, vv_sh.at[pl.ds(80 + tid * L, L)])

        plsc.subcore_barrier()

        @pl.when(tid < NG)
        def _phase_b():
            pltpu.sync_copy(vv_sh, vv_v)
            pltpu.sync_copy(x_hbm.at[pl.ds(10 * 80, 80)], x10_v)
            pltpu.sync_copy(x_hbm.at[pl.ds(tid * L * 80, L * 80)],
                            xslab_v)

            sacc = jnp.zeros((L,), jnp.float32)
            for g in range(NG):
                sacc = sacc + x10_v[pl.ds(g * L, L)] * vv_v[pl.ds(g * L, L)]
            s_vec = _lane_allsum(sacc)

            rowbase = lax.iota(jnp.int32, L) * 80
            acc = jnp.zeros((L,), jnp.float32)
            for blk in range(NG):
                coefs = vv_v[pl.ds(80 + blk * L, L)]
                for l in range(L):
                    k = blk * L + l
                    col = plsc.load_gather(xslab_v, [rowbase + k])
                    acc = acc + coefs[l] * col
            t = acc + s_vec
            est_v[...] = jnp.where(t >= 0.0, t, 0.2 * t)
            pltpu.sync_copy(est_v, e_sh.at[pl.ds(tid * L, L)])

        plsc.subcore_barrier()

        @pl.when(tid == 0)
        def _drain():
            pltpu.sync_copy(e_sh.at[pl.ds(0, ROWS)], out77_v)
            pltpu.sync_copy(out77_v, out_hbm)


@functools.partial(
    pl.kernel,
    out_type=jax.ShapeDtypeStruct((ROWS,), jnp.float32),
    mesh=plsc.VectorSubcoreMesh(core_axis_name="c", subcore_axis_name="s"),
    scratch_types=[
        pltpu.VMEM((208,), jnp.float32),
        pltpu.VMEM((NO, 80), jnp.float32),
        pltpu.VMEM((L * 80,), jnp.float32),
        pltpu.VMEM((80,), jnp.float32),
        pltpu.VMEM((2 * 80,), jnp.float32),
        pltpu.VMEM((L,), jnp.float32),
        pltpu.VMEM((L,), jnp.float32),
        pltpu.VMEM((ROWS,), jnp.float32),
        pltpu.VMEM_SHARED((2 * 80,), jnp.float32),
        pltpu.VMEM_SHARED((RPAD,), jnp.float32),
    ],
    compiler_params=pltpu.CompilerParams(needs_layout_passes=False),
    name="similarity_sc",
)
def _similarity_sc(x_hbm, w_hbm, a_hbm, out_hbm, *scratch):
    _body(x_hbm, w_hbm, a_hbm, out_hbm, *scratch)


def kernel(chicago_region_representations, W_lin, W_attn):
    x = jnp.asarray(chicago_region_representations, jnp.float32)
    xp = jnp.zeros((RPAD, 80), jnp.float32).at[:ROWS].set(x)
    e = _similarity_sc(xp.reshape(-1), W_lin.astype(jnp.float32),
                       W_attn.astype(jnp.float32))
    return e.reshape(ROWS, 1)


# speedup vs baseline: 1.0534x; 1.0534x over previous
import functools

import jax
import jax.numpy as jnp
from jax import lax
from jax.experimental import pallas as pl
from jax.experimental.pallas import tpu as pltpu
from jax.experimental.pallas import tpu_sc as plsc


def _lane_allsum(v):
    idx = lax.iota(jnp.int32, 16)
    for sh in (8, 4, 2, 1):
        v = v + v.at[idx ^ sh].get(mode="promise_in_bounds")
    return v


L = 16
NG = 5
ROWS = 77
RPAD = 80
NO = 90
XW = RPAD * 80
WBASE = XW
ABASE = XW + NO * 80
PACK = ABASE + 2 * NO + 16


def _body(pack_hbm, out_hbm, wa_v, xs_v, x10_v, vv_v, st2_v, est_v, sem,
          vv_sh):
    c = lax.axis_index("c")
    tid = lax.axis_index("s")

    @pl.when(c == 0)
    def _core0():
        @pl.when(tid < NG)
        def _load_and_phase_a():
            g = tid
            cp1 = pltpu.async_copy(
                pack_hbm.at[pl.ds(WBASE, NO * 80 + 2 * NO + 16)], wa_v, sem)
            cp2 = pltpu.async_copy(
                pack_hbm.at[pl.ds(g * L * 80, L * 80)], xs_v, sem)
            cp3 = pltpu.async_copy(pack_hbm.at[pl.ds(10 * 80, 80)], x10_v, sem)
            cp1.wait()
            cp2.wait()
            cp3.wait()

            AOFF = NO * 80
            acc1 = jnp.zeros((L,), jnp.float32)
            acc2 = jnp.zeros((L,), jnp.float32)
            for blk in range(6):
                coefs1 = wa_v[pl.ds(AOFF + blk * L, L)]
                coefs2 = wa_v[pl.ds(AOFF + NO + blk * L, L)]
                for l in range(L):
                    o = blk * L + l
                    if o >= NO:
                        break
                    wrow = wa_v[pl.ds(o * 80 + g * L, L)]
                    acc1 = acc1 + coefs1[l] * wrow
                    acc2 = acc2 + coefs2[l] * wrow
            st2_v[pl.ds(0, L)] = acc1
            st2_v[pl.ds(L, L)] = acc2
            pltpu.sync_copy(st2_v, vv_sh.at[pl.ds(tid * 2 * L, 2 * L)])

        plsc.subcore_barrier()

        @pl.when(tid < NG)
        def _phase_b():
            pltpu.sync_copy(vv_sh, vv_v)

            sacc = jnp.zeros((L,), jnp.float32)
            for g in range(NG):
                sacc = sacc + (x10_v[pl.ds(g * L, L)]
                               * vv_v[pl.ds((2 * g) * L, L)])
            s_vec = _lane_allsum(sacc)

            rowbase = lax.iota(jnp.int32, L) * 80
            acc = jnp.zeros((L,), jnp.float32)
            for blk in range(NG):
                coefs = vv_v[pl.ds((2 * blk + 1) * L, L)]
                for l in range(L):
                    k = blk * L + l
                    col = plsc.load_gather(xs_v, [rowbase + k])
                    acc = acc + coefs[l] * col
            t = acc + s_vec
            est_v[...] = jnp.where(t >= 0.0, t, 0.2 * t)
            pltpu.sync_copy(est_v, out_hbm.at[pl.ds(tid * L, L)])


@functools.partial(
    pl.kernel,
    out_type=jax.ShapeDtypeStruct((RPAD,), jnp.float32),
    mesh=plsc.VectorSubcoreMesh(core_axis_name="c", subcore_axis_name="s"),
    scratch_types=[
        pltpu.VMEM((NO * 80 + 2 * NO + 16,), jnp.float32),
        pltpu.VMEM((L * 80,), jnp.float32),
        pltpu.VMEM((80,), jnp.float32),
        pltpu.VMEM((2 * 80,), jnp.float32),
        pltpu.VMEM((2 * L,), jnp.float32),
        pltpu.VMEM((L,), jnp.float32),
        pltpu.SemaphoreType.DMA,
        pltpu.VMEM_SHARED((2 * 80,), jnp.float32),
    ],
    compiler_params=pltpu.CompilerParams(needs_layout_passes=False),
    name="similarity_sc",
)
def _similarity_sc(pack_hbm, out_hbm, *scratch):
    _body(pack_hbm, out_hbm, *scratch)


def kernel(chicago_region_representations, W_lin, W_attn):
    x = jnp.asarray(chicago_region_representations, jnp.float32)
    xp = jnp.zeros((RPAD, 80), jnp.float32).at[:ROWS].set(x)
    pack = jnp.concatenate([
        xp.reshape(-1),
        W_lin.astype(jnp.float32).reshape(-1),
        W_attn.astype(jnp.float32).reshape(-1),
        jnp.zeros((16,), jnp.float32),
    ])
    e = _similarity_sc(pack)
    return e[:ROWS].reshape(ROWS, 1)


# speedup vs baseline: 1.1187x; 1.0619x over previous
import functools

import jax
import jax.numpy as jnp
from jax import lax
from jax.experimental import pallas as pl
from jax.experimental.pallas import tpu as pltpu
from jax.experimental.pallas import tpu_sc as plsc


def _lane_allsum(v):
    idx = lax.iota(jnp.int32, 16)
    for sh in (8, 4, 2, 1):
        v = v + v.at[idx ^ sh].get(mode="promise_in_bounds")
    return v


L = 16
NG = 5
ROWS = 77
RPAD = 80
NO = 90
XW = RPAD * 80
WBASE = XW
ABASE = XW + NO * 80
PACK = ABASE + 2 * NO + 16


def _body(pack_hbm, out_hbm, wa_v, xs_v, x10_v, vv_v, st2_v, est_v, sem,
          vv_sh):
    c = lax.axis_index("c")
    tid = lax.axis_index("s")

    @pl.when(c == 0)
    def _core0():
        @pl.when(tid < NG)
        def _load_and_phase_a():
            g = tid
            cp1 = pltpu.async_copy(
                pack_hbm.at[pl.ds(WBASE, NO * 80 + 2 * NO + 16)], wa_v, sem)
            cp2 = pltpu.async_copy(
                pack_hbm.at[pl.ds(g * L * 80, L * 80)], xs_v, sem)
            cp3 = pltpu.async_copy(pack_hbm.at[pl.ds(10 * 80, 80)], x10_v, sem)
            cp1.wait()
            cp2.wait()
            cp3.wait()

            AOFF = NO * 80
            acc1 = jnp.zeros((L,), jnp.float32)
            acc2 = jnp.zeros((L,), jnp.float32)
            for blk in range(6):
                coefs1 = wa_v[pl.ds(AOFF + blk * L, L)]
                coefs2 = wa_v[pl.ds(AOFF + NO + blk * L, L)]
                for l in range(L):
                    o = blk * L + l
                    if o >= NO:
                        break
                    wrow = wa_v[pl.ds(o * 80 + g * L, L)]
                    acc1 = acc1 + coefs1[l] * wrow
                    acc2 = acc2 + coefs2[l] * wrow
            st2_v[pl.ds(0, L)] = acc1
            st2_v[pl.ds(L, L)] = acc2
            pltpu.sync_copy(st2_v, vv_sh.at[pl.ds(tid * 2 * L, 2 * L)])

        plsc.subcore_barrier()

        @pl.when(tid < NG)
        def _phase_b():
            pltpu.sync_copy(vv_sh, vv_v)

            sacc = jnp.zeros((L,), jnp.float32)
            for g in range(NG):
                sacc = sacc + (x10_v[pl.ds(g * L, L)]
                               * vv_v[pl.ds((2 * g) * L, L)])
            s_vec = _lane_allsum(sacc)

            rowbase = lax.iota(jnp.int32, L) * 80
            acc = jnp.zeros((L,), jnp.float32)
            for blk in range(NG):
                coefs = vv_v[pl.ds((2 * blk + 1) * L, L)]
                for l in range(L):
                    k = blk * L + l
                    col = plsc.load_gather(xs_v, [rowbase + k])
                    acc = acc + coefs[l] * col
            t = acc + s_vec
            est_v[...] = jnp.where(t >= 0.0, t, 0.2 * t)
            pltpu.sync_copy(est_v, out_hbm.at[pl.ds(tid * L, L)])


@functools.partial(
    pl.kernel,
    out_type=jax.ShapeDtypeStruct((RPAD,), jnp.float32),
    mesh=plsc.VectorSubcoreMesh(core_axis_name="c", subcore_axis_name="s",
                                num_cores=1),
    scratch_types=[
        pltpu.VMEM((NO * 80 + 2 * NO + 16,), jnp.float32),
        pltpu.VMEM((L * 80,), jnp.float32),
        pltpu.VMEM((80,), jnp.float32),
        pltpu.VMEM((2 * 80,), jnp.float32),
        pltpu.VMEM((2 * L,), jnp.float32),
        pltpu.VMEM((L,), jnp.float32),
        pltpu.SemaphoreType.DMA,
        pltpu.VMEM_SHARED((2 * 80,), jnp.float32),
    ],
    compiler_params=pltpu.CompilerParams(needs_layout_passes=False),
    name="similarity_sc",
)
def _similarity_sc(pack_hbm, out_hbm, *scratch):
    _body(pack_hbm, out_hbm, *scratch)


def kernel(chicago_region_representations, W_lin, W_attn):
    x = jnp.asarray(chicago_region_representations, jnp.float32)
    xp = jnp.zeros((RPAD, 80), jnp.float32).at[:ROWS].set(x)
    pack = jnp.concatenate([
        xp.reshape(-1),
        W_lin.astype(jnp.float32).reshape(-1),
        W_attn.astype(jnp.float32).reshape(-1),
        jnp.zeros((16,), jnp.float32),
    ])
    e = _similarity_sc(pack)
    return e[:ROWS].reshape(ROWS, 1)


# speedup vs baseline: 1.1412x; 1.0201x over previous
import functools

import jax
import jax.numpy as jnp
from jax import lax
from jax.experimental import pallas as pl
from jax.experimental.pallas import tpu as pltpu
from jax.experimental.pallas import tpu_sc as plsc


def _lane_allsum(v):
    idx = lax.iota(jnp.int32, 16)
    for sh in (8, 4, 2, 1):
        v = v + v.at[idx ^ sh].get(mode="promise_in_bounds")
    return v


L = 16
NG = 5
ROWS = 77
RPAD = 80
NO = 90
XTB = 0
WB = RPAD * 80
AX = WB + NG * NO * L


def _body(pack_hbm, out_hbm, xt_v, wb_v, ax_v, vv_v, st2_v, est_v, sem,
          vv_sh):
    tid = lax.axis_index("s")

    @pl.when(tid < NG)
    def _load_and_phase_a():
        g = tid
        cp1 = pltpu.async_copy(
            pack_hbm.at[pl.ds(XTB + g * (80 * L), 80 * L)], xt_v, sem)
        cp2 = pltpu.async_copy(
            pack_hbm.at[pl.ds(WB + g * (NO * L), NO * L)], wb_v, sem)
        cp3 = pltpu.async_copy(
            pack_hbm.at[pl.ds(AX, 2 * NO + RPAD)], ax_v, sem)
        cp1.wait()
        cp2.wait()
        cp3.wait()

        acc1 = jnp.zeros((L,), jnp.float32)
        acc2 = jnp.zeros((L,), jnp.float32)
        for blk in range(6):
            coefs1 = ax_v[pl.ds(blk * L, L)]
            coefs2 = ax_v[pl.ds(NO + blk * L, L)]
            for l in range(L):
                o = blk * L + l
                if o >= NO:
                    break
                wrow = wb_v[pl.ds(o * L, L)]
                acc1 = acc1 + coefs1[l] * wrow
                acc2 = acc2 + coefs2[l] * wrow
        st2_v[pl.ds(0, L)] = acc1
        st2_v[pl.ds(L, L)] = acc2
        pltpu.sync_copy(st2_v, vv_sh.at[pl.ds(tid * 2 * L, 2 * L)])

    plsc.subcore_barrier()

    @pl.when(tid < NG)
    def _phase_b():
        pltpu.sync_copy(vv_sh, vv_v)

        sacc = jnp.zeros((L,), jnp.float32)
        for g in range(NG):
            sacc = sacc + (ax_v[pl.ds(2 * NO + g * L, L)]
                           * vv_v[pl.ds((2 * g) * L, L)])
        s_vec = _lane_allsum(sacc)

        acc = jnp.zeros((L,), jnp.float32)
        for blk in range(NG):
            coefs = vv_v[pl.ds((2 * blk + 1) * L, L)]
            for l in range(L):
                k = blk * L + l
                acc = acc + coefs[l] * xt_v[pl.ds(k * L, L)]
        t = acc + s_vec
        est_v[...] = jnp.where(t >= 0.0, t, 0.2 * t)
        pltpu.sync_copy(est_v, out_hbm.at[pl.ds(tid * L, L)])


@functools.partial(
    pl.kernel,
    out_type=jax.ShapeDtypeStruct((RPAD,), jnp.float32),
    mesh=plsc.VectorSubcoreMesh(core_axis_name="c", subcore_axis_name="s",
                                num_cores=1),
    scratch_types=[
        pltpu.VMEM((80 * L,), jnp.float32),
        pltpu.VMEM((NO * L,), jnp.float32),
        pltpu.VMEM((2 * NO + RPAD,), jnp.float32),
        pltpu.VMEM((2 * 80,), jnp.float32),
        pltpu.VMEM((2 * L,), jnp.float32),
        pltpu.VMEM((L,), jnp.float32),
        pltpu.SemaphoreType.DMA,
        pltpu.VMEM_SHARED((2 * 80,), jnp.float32),
    ],
    compiler_params=pltpu.CompilerParams(needs_layout_passes=False),
    name="similarity_sc",
)
def _similarity_sc(pack_hbm, out_hbm, *scratch):
    _body(pack_hbm, out_hbm, *scratch)


def kernel(chicago_region_representations, W_lin, W_attn):
    x = jnp.asarray(chicago_region_representations, jnp.float32)
    xp = jnp.zeros((RPAD, 80), jnp.float32).at[:ROWS].set(x)
    xtb = xp.T.reshape(80, NG, L).transpose(1, 0, 2)
    wb = W_lin.astype(jnp.float32).reshape(NO, NG, L).transpose(1, 0, 2)
    pack = jnp.concatenate([
        xtb.reshape(-1),
        wb.reshape(-1),
        W_attn.astype(jnp.float32).reshape(-1),
        x[10],
    ])
    e = _similarity_sc(pack)
    return e[:ROWS].reshape(ROWS, 1)
